# Initial kernel scaffold; baseline (speedup 1.0000x reference)
import jax
import jax.numpy as jnp
from jax.experimental import pallas as pl


def kernel(indices, embedding_table):
    raise NotImplementedError("write your pallas kernel here")


# speedup vs baseline: 1.4372x; 1.4372x over previous
import functools

import jax
import jax.numpy as jnp
from jax import lax
from jax.experimental import pallas as pl
from jax.experimental.pallas import tpu as pltpu
from jax.experimental.pallas import tpu_sc as plsc

_BATCH = 16384
_N_FIELDS = 26
_OUT_DIM = 32
_TOTAL = _BATCH * _N_FIELDS

_NC = 2
_NS = 16
_NW = _NC * _NS
_PER_W = _TOTAL // _NW
_C = 128
_K = _PER_W // _C

assert _PER_W * _NW == _TOTAL
assert _K * _C == _PER_W


@jax.jit
def _sc_gather(idx2d, table):
    mesh = plsc.VectorSubcoreMesh(core_axis_name="c", subcore_axis_name="s")

    @functools.partial(
        pl.kernel,
        out_type=jax.ShapeDtypeStruct((_TOTAL, _OUT_DIM), jnp.float32),
        mesh=mesh,
        compiler_params=pltpu.CompilerParams(use_tc_tiling_on_sc=False),
        scratch_types=[
            pltpu.VMEM((_K, _C), jnp.int32),
            pltpu.VMEM((_C, _OUT_DIM), jnp.float32),
            pltpu.SemaphoreType.DMA,
        ],
    )
    def body(idx_hbm, table_hbm, out_hbm, idx_v, rows_v, sem):
        wid = lax.axis_index("s") * _NC + lax.axis_index("c")
        pltpu.sync_copy(idx_hbm.at[pl.ds(wid * _K, _K)], idx_v)
        base = wid * _PER_W

        def step(j, carry):
            pltpu.async_copy(table_hbm.at[idx_v.at[j]], rows_v, sem).wait()
            pltpu.sync_copy(rows_v, out_hbm.at[pl.ds(base + j * _C, _C)])
            return carry

        lax.fori_loop(0, _K, step, 0)

    return body(idx2d, table)


def kernel(indices, embedding_table):
    idx2d = indices.astype(jnp.int32).reshape(_TOTAL // _C, _C)
    out = _sc_gather(idx2d, embedding_table)
    return out.reshape(_BATCH, _N_FIELDS, _OUT_DIM)


# speedup vs baseline: 1.5536x; 1.0810x over previous
import functools

import jax
import jax.numpy as jnp
from jax import lax
from jax.experimental import pallas as pl
from jax.experimental.pallas import tpu as pltpu
from jax.experimental.pallas import tpu_sc as plsc

_BATCH = 16384
_N_FIELDS = 26
_OUT_DIM = 32
_TOTAL = _BATCH * _N_FIELDS

_NC = 2
_NS = 16
_NW = _NC * _NS
_PER_W = _TOTAL // _NW
_C = 832
_K = _PER_W // _C

assert _PER_W * _NW == _TOTAL
assert _K * _C == _PER_W


@jax.jit
def _sc_gather(idx2d, table):
    mesh = plsc.VectorSubcoreMesh(core_axis_name="c", subcore_axis_name="s")

    @functools.partial(
        pl.kernel,
        out_type=jax.ShapeDtypeStruct((_TOTAL, _OUT_DIM), jnp.float32),
        mesh=mesh,
        compiler_params=pltpu.CompilerParams(use_tc_tiling_on_sc=False),
        scratch_types=[
            pltpu.VMEM((_K, _C), jnp.int32),
            pltpu.VMEM((_C, _OUT_DIM), jnp.float32),
            pltpu.SemaphoreType.DMA,
        ],
    )
    def body(idx_hbm, table_hbm, out_hbm, idx_v, rows_v, sem):
        wid = lax.axis_index("s") * _NC + lax.axis_index("c")
        pltpu.sync_copy(idx_hbm.at[pl.ds(wid * _K, _K)], idx_v)
        base = wid * _PER_W

        def step(j, carry):
            pltpu.async_copy(table_hbm.at[idx_v.at[j]], rows_v, sem).wait()
            pltpu.sync_copy(rows_v, out_hbm.at[pl.ds(base + j * _C, _C)])
            return carry

        lax.fori_loop(0, _K, step, 0)

    return body(idx2d, table)


def kernel(indices, embedding_table):
    idx2d = indices.astype(jnp.int32).reshape(_TOTAL // _C, _C)
    out = _sc_gather(idx2d, embedding_table)
    return out.reshape(_BATCH, _N_FIELDS, _OUT_DIM)


# speedup vs baseline: 1.5677x; 1.0091x over previous
import functools

import jax
import jax.numpy as jnp
from jax import lax
from jax.experimental import pallas as pl
from jax.experimental.pallas import tpu as pltpu
from jax.experimental.pallas import tpu_sc as plsc

_BATCH = 16384
_N_FIELDS = 26
_OUT_DIM = 32
_TOTAL = _BATCH * _N_FIELDS

_NC = 2
_NS = 16
_NW = _NC * _NS
_PER_W = _TOTAL // _NW
_C = 832
_K = _PER_W // _C
_H = 4
_G = _K // _H

assert _PER_W * _NW == _TOTAL
assert _K * _C == _PER_W
assert _G * _H == _K


@jax.jit
def _sc_gather(idx2d, table):
    mesh = plsc.VectorSubcoreMesh(core_axis_name="c", subcore_axis_name="s")

    @functools.partial(
        pl.kernel,
        out_type=jax.ShapeDtypeStruct((_TOTAL, _OUT_DIM), jnp.float32),
        mesh=mesh,
        compiler_params=pltpu.CompilerParams(use_tc_tiling_on_sc=False),
        scratch_types=(
            [pltpu.VMEM((_K, _C), jnp.int32), pltpu.VMEM((_H, _C, _OUT_DIM), jnp.float32)]
            + [pltpu.SemaphoreType.DMA] * (2 * _H)
        ),
    )
    def body(idx_hbm, table_hbm, out_hbm, idx_v, rows_v, *sems):
        sem_g = sems[:_H]
        sem_o = sems[_H:]
        wid = lax.axis_index("s") * _NC + lax.axis_index("c")
        pltpu.sync_copy(idx_hbm.at[pl.ds(wid * _K, _K)], idx_v)
        base = wid * _PER_W

        def fire_gather(g, h):
            pltpu.async_copy(table_hbm.at[idx_v.at[g]], rows_v.at[h], sem_g[h])

        def wait_gather(h):
            pltpu.make_async_copy(table_hbm.at[idx_v.at[0]], rows_v.at[h], sem_g[h]).wait()

        def fire_out(g, h):
            pltpu.async_copy(rows_v.at[h], out_hbm.at[pl.ds(base + g * _C, _C)], sem_o[h])

        def wait_out(h):
            pltpu.make_async_copy(
                rows_v.at[h], out_hbm.at[pl.ds(base, _C)], sem_o[h]
            ).wait()

        for h in range(_H):
            fire_gather(h, h)

        def step(i, carry):
            g0 = i * _H
            for h in range(_H):
                wait_gather(h)
                fire_out(g0 + h, h)
            for h in range(_H):
                wait_out(h)

                @pl.when(i < _G - 1)
                def _():
                    fire_gather(g0 + h + _H, h)

            return carry

        lax.fori_loop(0, _G, step, 0)

    return body(idx2d, table)


def kernel(indices, embedding_table):
    idx2d = indices.astype(jnp.int32).reshape(_TOTAL // _C, _C)
    out = _sc_gather(idx2d, embedding_table)
    return out.reshape(_BATCH, _N_FIELDS, _OUT_DIM)
